# Initial kernel scaffold; baseline (speedup 1.0000x reference)
import jax
import jax.numpy as jnp
from jax.experimental import pallas as pl


def kernel(sample_indices, batch_size, labels, z_table, s_tissue_table):
    raise NotImplementedError("write your pallas kernel here")


# speedup vs baseline: 2.2094x; 2.2094x over previous
import functools

import jax
import jax.numpy as jnp
from jax import lax
from jax.experimental import pallas as pl
from jax.experimental.pallas import tpu as pltpu
from jax.experimental.pallas import tpu_sc as plsc


def _make_sc_kernel(B, DZ, DS, b_per_w, NC):
    mesh = plsc.VectorSubcoreMesh(core_axis_name="c", subcore_axis_name="s")

    @functools.partial(
        pl.kernel,
        mesh=mesh,
        out_type=(
            jax.ShapeDtypeStruct((B, DZ + DS), jnp.float32),
            jax.ShapeDtypeStruct((B, DZ), jnp.float32),
            jax.ShapeDtypeStruct((B, DS), jnp.float32),
        ),
        scratch_types=[
            pltpu.VMEM((b_per_w,), jnp.int32),
            pltpu.VMEM((b_per_w,), jnp.int32),
            pltpu.VMEM((b_per_w, DZ), jnp.float32),
            pltpu.VMEM((b_per_w, DS), jnp.float32),
            pltpu.SemaphoreType.DMA,
            pltpu.SemaphoreType.DMA,
        ],
        compiler_params=pltpu.CompilerParams(use_tc_tiling_on_sc=False),
    )
    def sc_kernel(zi_hbm, li_hbm, zt_hbm, st_hbm, tl_hbm, z_hbm, s_hbm,
                  zi_v, li_v, z_v, s_v, sem_z, sem_s):
        wid = lax.axis_index("s") * NC + lax.axis_index("c")
        base = wid * b_per_w
        pltpu.sync_copy(zi_hbm.at[pl.ds(base, b_per_w)], zi_v)
        pltpu.sync_copy(li_hbm.at[pl.ds(base, b_per_w)], li_v)
        cz = pltpu.async_copy(zt_hbm.at[zi_v], z_v, sem_z)
        cs = pltpu.async_copy(st_hbm.at[li_v], s_v, sem_s)
        cz.wait()
        cs.wait()
        pltpu.sync_copy(z_v, z_hbm.at[pl.ds(base, b_per_w)])
        pltpu.sync_copy(s_v, s_hbm.at[pl.ds(base, b_per_w)])
        pltpu.sync_copy(z_v, tl_hbm.at[pl.ds(base, b_per_w), pl.ds(0, DZ)])
        pltpu.sync_copy(s_v, tl_hbm.at[pl.ds(base, b_per_w), pl.ds(DZ, DS)])

    return sc_kernel


def kernel(sample_indices, batch_size, labels, z_table, s_tissue_table):
    B = sample_indices.shape[0]
    DZ = z_table.shape[1]
    DS = s_tissue_table.shape[1]
    info = plsc.get_sparse_core_info()
    NC, NS = info.num_cores, info.num_subcores
    NW = NC * NS
    b_per_w = B // NW

    zi = sample_indices.astype(jnp.int32)
    li = labels[:, 0].astype(jnp.int32)

    sc_kernel = _make_sc_kernel(B, DZ, DS, b_per_w, NC)
    total_latent, z, s = sc_kernel(zi, li, z_table, s_tissue_table)
    return (total_latent, z, s)
